# baseline (speedup 1.0000x reference)
import functools

import jax
import jax.numpy as jnp
from jax import lax
from jax.experimental import pallas as pl
from jax.experimental.pallas import tpu as pltpu
from jax.experimental.pallas import tpu_sc as plsc

_TILE = 32
_NBUF = 2
_STREAM_FRAC_NUM = 3
_STREAM_FRAC_DEN = 4


def _sc_broadcast(pos_table, batch, seq, d):
    info = plsc.get_sparse_core_info()
    nw = info.num_cores * info.num_subcores
    ch_src = seq // nw
    ch_st = (ch_src * _STREAM_FRAC_NUM // _STREAM_FRAC_DEN) // _TILE * _TILE
    ch_sp = ch_src - ch_st
    nt_st = ch_st // _TILE
    nt_sp = ch_sp // _TILE
    mesh = plsc.VectorSubcoreMesh(core_axis_name="c", subcore_axis_name="s")

    @functools.partial(
        pl.kernel,
        mesh=mesh,
        out_type=jax.ShapeDtypeStruct((batch * seq, d), pos_table.dtype),
        scratch_types=[
            pltpu.VMEM((_NBUF, _TILE, d), pos_table.dtype),
            pltpu.VMEM_SHARED((info.num_subcores, _NBUF, _TILE, d), pos_table.dtype),
            pltpu.SemaphoreType.DMA((_NBUF,)),
            pltpu.SemaphoreType.DMA((_NBUF,)),
            pltpu.SemaphoreType.DMA((_NBUF,)),
            pltpu.SemaphoreType.DMA((_NBUF,)),
        ],
    )
    def run(table_hbm, out_hbm, buf, shared, in_st, out_st, in_sp, out_sp):
        sid = lax.axis_index("s")
        wid = sid * info.num_cores + lax.axis_index("c")
        src0 = wid * ch_src
        spm0 = src0 + ch_st

        def st_in(t, slot):
            return pltpu.make_async_copy(
                table_hbm.at[pl.ds(src0 + t * _TILE, _TILE)],
                buf.at[slot],
                in_st.at[slot],
            )

        def st_out(t, slot, b):
            return pltpu.make_async_copy(
                buf.at[slot],
                out_hbm.at[pl.ds(b * seq + src0 + t * _TILE, _TILE)],
                out_st.at[slot],
            )

        def sp_in(t, slot):
            return pltpu.make_async_copy(
                table_hbm.at[pl.ds(spm0 + t * _TILE, _TILE)],
                shared.at[sid, slot],
                in_sp.at[slot],
            )

        def sp_out(t, slot, b):
            return pltpu.make_async_copy(
                shared.at[sid, slot],
                out_hbm.at[pl.ds(b * seq + spm0 + t * _TILE, _TILE)],
                out_sp.at[slot],
            )

        st_in(0, 0).start()
        sp_in(0, 0).start()
        nt = max(nt_st, nt_sp)
        for t in range(nt):
            slot = t % _NBUF
            nslot = (t + 1) % _NBUF
            for n, in_copy, out_copy in (
                (nt_st, st_in, st_out),
                (nt_sp, sp_in, sp_out),
            ):
                if t >= n:
                    continue
                in_copy(t, slot).wait()
                for b in range(batch):
                    out_copy(t, slot, b).start()
                if t + 1 < n:
                    if t + 1 >= _NBUF:
                        for b in range(batch):
                            out_copy(t + 1 - _NBUF, nslot, b).wait()
                    in_copy(t + 1, nslot).start()
        for n, out_copy in ((nt_st, st_out), (nt_sp, sp_out)):
            for t in range(max(0, n - _NBUF + 1), n):
                for b in range(batch):
                    out_copy(t, t % _NBUF, b).wait()

    return run(pos_table).reshape(batch, seq, d)


def kernel(x, pos_table):
    batch, seq, _ = x.shape
    d = pos_table.shape[1]
    return _sc_broadcast(pos_table, batch, seq, d)


# speedup vs baseline: 1.0050x; 1.0050x over previous
import functools

import jax
from jax import lax
from jax.experimental import pallas as pl
from jax.experimental.pallas import tpu as pltpu
from jax.experimental.pallas import tpu_sc as plsc

_TILE = 32
_NBUF = 2
_STREAM_FRAC_NUM = 5
_STREAM_FRAC_DEN = 8


def _sc_broadcast(pos_table, batch, seq, d):
    info = plsc.get_sparse_core_info()
    nw = info.num_cores * info.num_subcores
    ch_src = seq // nw
    ch_st = (ch_src * _STREAM_FRAC_NUM // _STREAM_FRAC_DEN) // _TILE * _TILE
    ch_sp = ch_src - ch_st
    nt_st = ch_st // _TILE
    nt_sp = ch_sp // _TILE
    mesh = plsc.VectorSubcoreMesh(core_axis_name="c", subcore_axis_name="s")

    @functools.partial(
        pl.kernel,
        mesh=mesh,
        out_type=jax.ShapeDtypeStruct((batch * seq, d), pos_table.dtype),
        scratch_types=[
            pltpu.VMEM((_NBUF, _TILE, d), pos_table.dtype),
            pltpu.VMEM_SHARED((info.num_subcores, _NBUF, _TILE, d), pos_table.dtype),
            pltpu.SemaphoreType.DMA((_NBUF,)),
            pltpu.SemaphoreType.DMA((_NBUF,)),
            pltpu.SemaphoreType.DMA((_NBUF,)),
            pltpu.SemaphoreType.DMA((_NBUF,)),
        ],
    )
    def run(table_hbm, out_hbm, buf, shared, in_st, out_st, in_sp, out_sp):
        sid = lax.axis_index("s")
        wid = sid * info.num_cores + lax.axis_index("c")
        src0 = wid * ch_src
        spm0 = src0 + ch_st

        def st_in(t, slot):
            return pltpu.make_async_copy(
                table_hbm.at[pl.ds(src0 + t * _TILE, _TILE)],
                buf.at[slot],
                in_st.at[slot],
            )

        def st_out(t, slot, b):
            return pltpu.make_async_copy(
                buf.at[slot],
                out_hbm.at[pl.ds(b * seq + src0 + t * _TILE, _TILE)],
                out_st.at[slot],
            )

        def sp_in(t, slot):
            return pltpu.make_async_copy(
                table_hbm.at[pl.ds(spm0 + t * _TILE, _TILE)],
                shared.at[sid, slot],
                in_sp.at[slot],
            )

        def sp_out(t, slot, b):
            return pltpu.make_async_copy(
                shared.at[sid, slot],
                out_hbm.at[pl.ds(b * seq + spm0 + t * _TILE, _TILE)],
                out_sp.at[slot],
            )

        st_in(0, 0).start()
        sp_in(0, 0).start()
        nt = max(nt_st, nt_sp)
        for t in range(nt):
            slot = t % _NBUF
            nslot = (t + 1) % _NBUF
            for n, in_copy, out_copy in (
                (nt_st, st_in, st_out),
                (nt_sp, sp_in, sp_out),
            ):
                if t >= n:
                    continue
                in_copy(t, slot).wait()
                for b in range(batch):
                    out_copy(t, slot, b).start()
                if t + 1 < n:
                    if t + 1 >= _NBUF:
                        for b in range(batch):
                            out_copy(t + 1 - _NBUF, nslot, b).wait()
                    in_copy(t + 1, nslot).start()
        for n, out_copy in ((nt_st, st_out), (nt_sp, sp_out)):
            for t in range(max(0, n - _NBUF + 1), n):
                for b in range(batch):
                    out_copy(t, t % _NBUF, b).wait()

    return run(pos_table).reshape(batch, seq, d)


def kernel(x, pos_table):
    batch, seq, _ = x.shape
    d = pos_table.shape[1]
    return _sc_broadcast(pos_table, batch, seq, d)
